# baseline (speedup 1.0000x reference)
import functools

import jax
import jax.numpy as jnp
from jax import lax
from jax.experimental import pallas as pl
from jax.experimental.pallas import tpu as pltpu
from jax.experimental.pallas import tpu_sc as plsc

N = 10000
E = 160000
D = 256
OUT = 2
HF = 128
NP = 10240
TILES = 16
TPER = E // TILES
TPE = 10080
CH = 96
NCH = TPE // CH
GC = 21
NG = NCH // GC
EPAD = TILES * TPE - E
NSL = NP // TILES
NP2 = 10112
SPT = NP2 // TILES
ZR = 8

_MESH = plsc.VectorSubcoreMesh(
    core_axis_name="c", subcore_axis_name="s", num_cores=2, num_subcores=16)


@functools.partial(
    pl.kernel,
    out_type=jax.ShapeDtypeStruct((2, NP), jnp.float32),
    mesh=_MESH,
    scratch_types=[
        pltpu.VMEM((TPER,), jnp.int32),
        pltpu.VMEM((NP,), jnp.float32),
        pltpu.VMEM((TILES, NSL), jnp.float32),
        pltpu.VMEM((NSL,), jnp.float32),
        pltpu.VMEM_SHARED((TILES, NP), jnp.float32),
    ],
    compiler_params=pltpu.CompilerParams(needs_layout_passes=False),
)
def _sc_degrees(ei_hbm, deg_hbm, idx_v, cnt_v, tmp_v, acc_v, shared):
    c = lax.axis_index("c")
    s = lax.axis_index("s")
    pltpu.sync_copy(ei_hbm.at[c * TILES + s], idx_v)
    z16 = jnp.zeros((16,), jnp.float32)

    def zbody(j, carry):
        cnt_v[pl.ds(j * 16, 16)] = z16
        return carry
    lax.fori_loop(0, NP // 16, zbody, 0)

    ones = jnp.ones((16,), jnp.float32)

    def abody(j, carry):
        idx = idx_v[pl.ds(j * 16, 16)]
        plsc.addupdate_scatter(cnt_v, [idx], ones)
        return carry
    lax.fori_loop(0, TPER // 16, abody, 0)

    pltpu.sync_copy(cnt_v, shared.at[s])
    plsc.subcore_barrier()
    for r in range(TILES):
        pltpu.sync_copy(shared.at[r, pl.ds(s * NSL, NSL)], tmp_v.at[r])

    def rbody(j, carry):
        v = tmp_v[0, pl.ds(j * 16, 16)]
        for r in range(1, TILES):
            v = v + tmp_v[r, pl.ds(j * 16, 16)]
        acc_v[pl.ds(j * 16, 16)] = v
        return carry
    lax.fori_loop(0, NSL // 16, rbody, 0)
    pltpu.sync_copy(acc_v, deg_hbm.at[c, pl.ds(s * NSL, NSL)])


@functools.partial(
    pl.kernel,
    out_type=jax.ShapeDtypeStruct((2 * NP2, HF), jnp.float32),
    mesh=_MESH,
    scratch_types=[
        pltpu.VMEM((2, GC, CH), jnp.int32),
        pltpu.VMEM((2, GC, CH), jnp.int32),
        pltpu.VMEM((3, CH, HF), jnp.float32),
        pltpu.VMEM_SHARED((NP2, HF), jnp.float32),
        pltpu.SemaphoreType.DMA((3,)),
        pltpu.SemaphoreType.DMA((3,)),
        pltpu.SemaphoreType.DMA,
        pltpu.SemaphoreType.DMA,
    ],
    compiler_params=pltpu.CompilerParams(needs_layout_passes=False),
)
def _sc_spmm(h_hbm, src_hbm, dst_hbm, agg_hbm, src_m, dst_m, buf,
             agg_sh, gsem, ssem, zsem, isem):
    c = lax.axis_index("c")
    s = lax.axis_index("s")
    sbase = (c * TILES + s) * NG
    dbase = s * NG

    z16 = jnp.zeros((16,), jnp.float32)
    for r in range(ZR):
        for k in range(HF // 16):
            buf[0, r, pl.ds(k * 16, 16)] = z16
    zsrc = buf.at[0, pl.ds(0, ZR)]

    def zc(j, carry):
        pltpu.async_copy(zsrc, agg_sh.at[pl.ds(s * SPT + j * ZR, ZR)], zsem)

        @pl.when(j >= 4)
        def _():
            pltpu.make_async_copy(
                zsrc, agg_sh.at[pl.ds(s * SPT, ZR)], zsem).wait()
        return carry
    lax.fori_loop(0, SPT // ZR, zc, 0)
    for _ in range(4):
        pltpu.make_async_copy(zsrc, agg_sh.at[pl.ds(s * SPT, ZR)],
                              zsem).wait()
    plsc.subcore_barrier()

    pltpu.sync_copy(src_hbm.at[sbase], src_m.at[0])
    pltpu.sync_copy(dst_hbm.at[dbase], dst_m.at[0])
    pltpu.async_copy(src_hbm.at[sbase + 1], src_m.at[1], isem)
    pltpu.async_copy(dst_hbm.at[dbase + 1], dst_m.at[1], isem)
    pltpu.async_copy(h_hbm.at[src_m.at[0, 0]], buf.at[0], gsem.at[0])
    pltpu.async_copy(h_hbm.at[src_m.at[0, 1]], buf.at[1], gsem.at[1])

    for g in range(NG):
        cs, ns = g % 2, (g + 1) % 2
        cur_s, cur_d = src_m.at[cs], dst_m.at[cs]
        nxt_s, nxt_d = src_m.at[ns], dst_m.at[ns]
        off = g * GC

        def body(jl, carry):
            j = off + jl
            @pl.when(jnp.logical_and(j >= 1, j < NCH - 2))
            def _():
                q = lax.rem(j + 2, 3)
                rw = jnp.maximum(jl - 1, 0)
                pltpu.make_async_copy(
                    buf.at[q], agg_sh.at[cur_d.at[rw]], ssem.at[q]).wait()

            if 1 <= g <= NG - 2:
                @pl.when(jl == 0)
                def _():
                    pltpu.async_copy(src_hbm.at[sbase + g + 1], nxt_s, isem)
                    pltpu.async_copy(dst_hbm.at[dbase + g + 1], nxt_d, isem)

            @pl.when(jnp.logical_and(j < NCH - 2, jl < GC - 2))
            def _():
                p2 = lax.rem(j + 2, 3)
                pltpu.async_copy(
                    h_hbm.at[cur_s.at[jl + 2]], buf.at[p2], gsem.at[p2])
            if g < NG - 1:
                @pl.when(jl >= GC - 2)
                def _():
                    p2 = lax.rem(j + 2, 3)
                    pltpu.async_copy(
                        h_hbm.at[nxt_s.at[jl + 2 - GC]], buf.at[p2],
                        gsem.at[p2])

            pw = lax.rem(j, 3)
            pltpu.make_async_copy(
                h_hbm.at[cur_s.at[jl]], buf.at[pw], gsem.at[pw]).wait()
            pltpu.async_copy(
                buf.at[pw], agg_sh.at[cur_d.at[jl]], ssem.at[pw], add=True)
            return carry

        lax.fori_loop(0, GC - 2, body, 0)
        if g < NG - 1:
            pltpu.make_async_copy(src_hbm.at[sbase + g + 1], nxt_s,
                                  isem).wait()
            pltpu.make_async_copy(dst_hbm.at[dbase + g + 1], nxt_d,
                                  isem).wait()
        lax.fori_loop(GC - 2, GC, body, 0)

    lslot = (NG - 1) % 2
    for k in range(NCH - 3, NCH):
        q = k % 3
        pltpu.make_async_copy(
            buf.at[q], agg_sh.at[dst_m.at[lslot, k - (NG - 1) * GC]],
            ssem.at[q]).wait()

    plsc.subcore_barrier()
    pltpu.sync_copy(agg_sh.at[pl.ds(s * SPT, SPT)],
                    agg_hbm.at[pl.ds(c * NP2 + s * SPT, SPT)])


BN = 1000
GRID = N // BN


def _tc1_body(x_ref, w_ref, dg_ref, out_ref):
    r = lax.rsqrt(jnp.maximum(dg_ref[...], 1.0))
    y = jnp.dot(x_ref[...], w_ref[...], preferred_element_type=jnp.float32)
    y = y * r
    out_ref[0] = y[:, :HF]
    out_ref[1] = y[:, HF:]


_tc1 = pl.pallas_call(
    _tc1_body,
    grid=(GRID,),
    in_specs=[
        pl.BlockSpec((BN, D), lambda i: (i, 0)),
        pl.BlockSpec((D, D), lambda i: (0, 0)),
        pl.BlockSpec((BN, 1), lambda i: (i, 0)),
    ],
    out_specs=pl.BlockSpec((2, BN, HF), lambda i: (0, i, 0)),
    out_shape=jax.ShapeDtypeStruct((2, NP2, HF), jnp.float32),
)


def _tc2_body(a_ref, din_ref, dout_ref, w_ref, b_ref, out_ref):
    rin = lax.rsqrt(jnp.maximum(din_ref[...], 1.0))
    rout = lax.rsqrt(jnp.maximum(dout_ref[...], 1.0))
    a = jnp.concatenate([a_ref[0], a_ref[1]], axis=1)
    h = jnp.maximum(a * rin + b_ref[...], 0.0)
    y = jnp.dot(h, w_ref[...], preferred_element_type=jnp.float32) * rout
    out_ref[0] = y[:, :HF]
    out_ref[1] = y[:, HF:]


_tc2 = pl.pallas_call(
    _tc2_body,
    grid=(GRID,),
    in_specs=[
        pl.BlockSpec((2, BN, HF), lambda i: (0, i, 0)),
        pl.BlockSpec((BN, 1), lambda i: (i, 0)),
        pl.BlockSpec((BN, 1), lambda i: (i, 0)),
        pl.BlockSpec((D, D), lambda i: (0, 0)),
        pl.BlockSpec((1, D), lambda i: (0, 0)),
    ],
    out_specs=pl.BlockSpec((2, BN, HF), lambda i: (0, i, 0)),
    out_shape=jax.ShapeDtypeStruct((2, NP2, HF), jnp.float32),
)


def _tc3_body(a_ref, din_ref, b_ref, wr_ref, br_ref, out_ref, acc_ref):
    i = pl.program_id(0)

    @pl.when(i == 0)
    def _():
        acc_ref[...] = jnp.zeros_like(acc_ref)

    rin = lax.rsqrt(jnp.maximum(din_ref[...], 1.0))
    a = jnp.concatenate([a_ref[0], a_ref[1]], axis=1)
    h = jnp.maximum(a * rin + b_ref[...], 0.0)
    acc_ref[...] += jnp.sum(h, axis=0, keepdims=True)

    @pl.when(i == GRID - 1)
    def _():
        out_ref[...] = jnp.dot(
            acc_ref[...] * (1.0 / N), wr_ref[...],
            preferred_element_type=jnp.float32) + br_ref[...]


_tc3 = pl.pallas_call(
    _tc3_body,
    grid=(GRID,),
    in_specs=[
        pl.BlockSpec((2, BN, HF), lambda i: (0, i, 0)),
        pl.BlockSpec((BN, 1), lambda i: (i, 0)),
        pl.BlockSpec((1, D), lambda i: (0, 0)),
        pl.BlockSpec((D, OUT), lambda i: (0, 0)),
        pl.BlockSpec((1, OUT), lambda i: (0, 0)),
    ],
    out_specs=pl.BlockSpec((1, OUT), lambda i: (0, 0)),
    out_shape=jax.ShapeDtypeStruct((1, OUT), jnp.float32),
    scratch_shapes=[pltpu.VMEM((1, D), jnp.float32)],
)


def kernel(x, edge_index, W1, b1, W2, b2, Wr, br):
    ei32 = edge_index.reshape(2 * TILES, TPER)
    deg = _sc_degrees(ei32)
    deg_out_col = deg[0, :N].reshape(N, 1)
    deg_in_col = deg[1, :N].reshape(N, 1)

    src = jnp.concatenate(
        [edge_index[0], jnp.zeros((EPAD,), jnp.int32)])
    dst = jnp.concatenate(
        [edge_index[1],
         N + (jnp.arange(EPAD, dtype=jnp.int32) % (NP2 - N))])
    src3 = jnp.stack([src, src + NP2]).reshape(2 * TILES * NG, GC, CH)
    dst3 = dst.reshape(TILES * NG, GC, CH)

    t1 = _tc1(x, W1, deg_out_col)
    agg1 = _sc_spmm(t1.reshape(2 * NP2, HF), src3, dst3)
    t2 = _tc2(agg1.reshape(2, NP2, HF), deg_in_col, deg_out_col,
              W2, b1.reshape(1, D))
    agg2 = _sc_spmm(t2.reshape(2 * NP2, HF), src3, dst3)
    return _tc3(agg2.reshape(2, NP2, HF), deg_in_col,
                b2.reshape(1, D), Wr, br.reshape(1, OUT))


# speedup vs baseline: 1.3813x; 1.3813x over previous
import functools

import jax
import jax.numpy as jnp
from jax import lax
from jax.experimental import pallas as pl
from jax.experimental.pallas import tpu as pltpu
from jax.experimental.pallas import tpu_sc as plsc

N = 10000
E = 160000
D = 256
OUT = 2
HF = 128
NP = 10240
TILES = 16
TPE = E // TILES
CH = 125
NCH = TPE // CH
GC = 20
NG = NCH // GC
NSL = NP // TILES
NP2 = 10112
SPT = NP2 // TILES
ZR = 8

_MESH = plsc.VectorSubcoreMesh(
    core_axis_name="c", subcore_axis_name="s", num_cores=2, num_subcores=16)


@functools.partial(
    pl.kernel,
    out_type=jax.ShapeDtypeStruct((2, NP), jnp.float32),
    mesh=_MESH,
    scratch_types=[
        pltpu.VMEM((TPE,), jnp.int32),
        pltpu.VMEM((NP,), jnp.float32),
        pltpu.VMEM((TILES, NSL), jnp.float32),
        pltpu.VMEM((NSL,), jnp.float32),
        pltpu.VMEM_SHARED((TILES, NP), jnp.float32),
    ],
    compiler_params=pltpu.CompilerParams(needs_layout_passes=False),
)
def _sc_degrees(ei_hbm, deg_hbm, idx_v, cnt_v, tmp_v, acc_v, shared):
    c = lax.axis_index("c")
    s = lax.axis_index("s")
    pltpu.sync_copy(ei_hbm.at[c * TILES + s], idx_v)
    z16 = jnp.zeros((16,), jnp.float32)

    def zbody(j, carry):
        cnt_v[pl.ds(j * 16, 16)] = z16
        return carry
    lax.fori_loop(0, NP // 16, zbody, 0)

    ones = jnp.ones((16,), jnp.float32)

    def abody(j, carry):
        idx = idx_v[pl.ds(j * 16, 16)]
        plsc.addupdate_scatter(cnt_v, [idx], ones)
        return carry
    lax.fori_loop(0, TPE // 16, abody, 0)

    pltpu.sync_copy(cnt_v, shared.at[s])
    plsc.subcore_barrier()
    for r in range(TILES):
        pltpu.sync_copy(shared.at[r, pl.ds(s * NSL, NSL)], tmp_v.at[r])

    def rbody(j, carry):
        v = tmp_v[0, pl.ds(j * 16, 16)]
        for r in range(1, TILES):
            v = v + tmp_v[r, pl.ds(j * 16, 16)]
        acc_v[pl.ds(j * 16, 16)] = v
        return carry
    lax.fori_loop(0, NSL // 16, rbody, 0)
    pltpu.sync_copy(acc_v, deg_hbm.at[c, pl.ds(s * NSL, NSL)])


@functools.partial(
    pl.kernel,
    out_type=jax.ShapeDtypeStruct((2 * NP2, HF), jnp.float32),
    mesh=_MESH,
    scratch_types=[
        pltpu.VMEM((GC, CH), jnp.int32),
        pltpu.VMEM((GC, CH), jnp.int32),
        pltpu.VMEM((2, CH, HF), jnp.float32),
        pltpu.VMEM((ZR, HF), jnp.float32),
        pltpu.VMEM_SHARED((NP2, HF), jnp.float32),
        pltpu.SemaphoreType.DMA((2,)),
        pltpu.SemaphoreType.DMA,
    ],
    compiler_params=pltpu.CompilerParams(needs_layout_passes=False),
)
def _sc_spmm(h_hbm, src_hbm, dst_hbm, agg_hbm, src_m, dst_m, buf, zbuf,
             agg_sh, gsem, zsem):
    c = lax.axis_index("c")
    s = lax.axis_index("s")

    z16 = jnp.zeros((16,), jnp.float32)
    for r in range(ZR):
        for k in range(HF // 16):
            zbuf[r, pl.ds(k * 16, 16)] = z16

    def zc(j, carry):
        pltpu.async_copy(zbuf, agg_sh.at[pl.ds(s * SPT + j * ZR, ZR)], zsem)

        @pl.when(j >= 4)
        def _():
            pltpu.make_async_copy(
                zbuf, agg_sh.at[pl.ds(s * SPT, ZR)], zsem).wait()
        return carry
    lax.fori_loop(0, SPT // ZR, zc, 0)
    for _ in range(4):
        pltpu.make_async_copy(zbuf, agg_sh.at[pl.ds(s * SPT, ZR)],
                              zsem).wait()
    plsc.subcore_barrier()

    for g in range(NG):
        pltpu.sync_copy(src_hbm.at[(c * TILES + s) * NG + g], src_m)
        pltpu.sync_copy(dst_hbm.at[s * NG + g], dst_m)

        pltpu.async_copy(h_hbm.at[src_m.at[0]], buf.at[0], gsem.at[0])

        def body(j, carry):
            p = lax.rem(j, 2)
            q = lax.rem(j + 1, 2)
            pltpu.async_copy(h_hbm.at[src_m.at[j]], buf.at[p], gsem.at[p])
            pltpu.make_async_copy(
                h_hbm.at[src_m.at[j - 1]], buf.at[q], gsem.at[q]).wait()
            pltpu.sync_copy(buf.at[q], agg_sh.at[dst_m.at[j - 1]], add=True)
            return carry
        lax.fori_loop(1, GC, body, 0)

        lp = (GC - 1) % 2
        pltpu.make_async_copy(
            h_hbm.at[src_m.at[GC - 1]], buf.at[lp], gsem.at[lp]).wait()
        pltpu.sync_copy(buf.at[lp], agg_sh.at[dst_m.at[GC - 1]], add=True)

    plsc.subcore_barrier()
    pltpu.sync_copy(agg_sh.at[pl.ds(s * SPT, SPT)],
                    agg_hbm.at[pl.ds(c * NP2 + s * SPT, SPT)])


BN = 1000
GRID = N // BN


def _tc1_body(x_ref, w_ref, dg_ref, out_ref):
    r = lax.rsqrt(jnp.maximum(dg_ref[...], 1.0))
    y = jnp.dot(x_ref[...].astype(jnp.bfloat16),
                w_ref[...].astype(jnp.bfloat16),
                preferred_element_type=jnp.float32)
    y = y * r
    out_ref[0] = y[:, :HF]
    out_ref[1] = y[:, HF:]


_tc1 = pl.pallas_call(
    _tc1_body,
    grid=(GRID,),
    in_specs=[
        pl.BlockSpec((BN, D), lambda i: (i, 0)),
        pl.BlockSpec((D, D), lambda i: (0, 0)),
        pl.BlockSpec((BN, 1), lambda i: (i, 0)),
    ],
    out_specs=pl.BlockSpec((2, BN, HF), lambda i: (0, i, 0)),
    out_shape=jax.ShapeDtypeStruct((2, NP2, HF), jnp.float32),
)


def _tc2_body(a_ref, din_ref, dout_ref, w_ref, b_ref, out_ref):
    rin = lax.rsqrt(jnp.maximum(din_ref[...], 1.0))
    rout = lax.rsqrt(jnp.maximum(dout_ref[...], 1.0))
    a = jnp.concatenate([a_ref[0], a_ref[1]], axis=1)
    h = jnp.maximum(a * rin + b_ref[...], 0.0)
    y = jnp.dot(h.astype(jnp.bfloat16), w_ref[...].astype(jnp.bfloat16),
                preferred_element_type=jnp.float32) * rout
    out_ref[0] = y[:, :HF]
    out_ref[1] = y[:, HF:]


_tc2 = pl.pallas_call(
    _tc2_body,
    grid=(GRID,),
    in_specs=[
        pl.BlockSpec((2, BN, HF), lambda i: (0, i, 0)),
        pl.BlockSpec((BN, 1), lambda i: (i, 0)),
        pl.BlockSpec((BN, 1), lambda i: (i, 0)),
        pl.BlockSpec((D, D), lambda i: (0, 0)),
        pl.BlockSpec((1, D), lambda i: (0, 0)),
    ],
    out_specs=pl.BlockSpec((2, BN, HF), lambda i: (0, i, 0)),
    out_shape=jax.ShapeDtypeStruct((2, NP2, HF), jnp.float32),
)


def _tc3_body(a_ref, din_ref, b_ref, wr_ref, br_ref, out_ref, acc_ref):
    i = pl.program_id(0)

    @pl.when(i == 0)
    def _():
        acc_ref[...] = jnp.zeros_like(acc_ref)

    rin = lax.rsqrt(jnp.maximum(din_ref[...], 1.0))
    a = jnp.concatenate([a_ref[0], a_ref[1]], axis=1)
    h = jnp.maximum(a * rin + b_ref[...], 0.0)
    acc_ref[...] += jnp.sum(h, axis=0, keepdims=True)

    @pl.when(i == GRID - 1)
    def _():
        out_ref[...] = jnp.dot(
            acc_ref[...] * (1.0 / N), wr_ref[...],
            preferred_element_type=jnp.float32) + br_ref[...]


_tc3 = pl.pallas_call(
    _tc3_body,
    grid=(GRID,),
    in_specs=[
        pl.BlockSpec((2, BN, HF), lambda i: (0, i, 0)),
        pl.BlockSpec((BN, 1), lambda i: (i, 0)),
        pl.BlockSpec((1, D), lambda i: (0, 0)),
        pl.BlockSpec((D, OUT), lambda i: (0, 0)),
        pl.BlockSpec((1, OUT), lambda i: (0, 0)),
    ],
    out_specs=pl.BlockSpec((1, OUT), lambda i: (0, 0)),
    out_shape=jax.ShapeDtypeStruct((1, OUT), jnp.float32),
    scratch_shapes=[pltpu.VMEM((1, D), jnp.float32)],
)


def kernel(x, edge_index, W1, b1, W2, b2, Wr, br):
    ei32 = edge_index.reshape(2 * TILES, TPE)
    deg = _sc_degrees(ei32)
    deg_out_col = deg[0, :N].reshape(N, 1)
    deg_in_col = deg[1, :N].reshape(N, 1)

    src = edge_index[0]
    src3 = jnp.stack([src, src + NP2]).reshape(2 * TILES * NG, GC, CH)
    dst3 = edge_index[1].reshape(TILES * NG, GC, CH)

    t1 = _tc1(x, W1, deg_out_col)
    agg1 = _sc_spmm(t1.reshape(2 * NP2, HF), src3, dst3)
    t2 = _tc2(agg1.reshape(2, NP2, HF), deg_in_col, deg_out_col,
              W2, b1.reshape(1, D))
    agg2 = _sc_spmm(t2.reshape(2 * NP2, HF), src3, dst3)
    return _tc3(agg2.reshape(2, NP2, HF), deg_in_col,
                b2.reshape(1, D), Wr, br.reshape(1, OUT))


# speedup vs baseline: 1.4190x; 1.0273x over previous
import functools

import jax
import jax.numpy as jnp
from jax import lax
from jax.experimental import pallas as pl
from jax.experimental.pallas import tpu as pltpu
from jax.experimental.pallas import tpu_sc as plsc

N = 10000
E = 160000
D = 256
OUT = 2
HF = 128
NP = 10240
TILES = 16
TPE = E // TILES
CH = 125
NCH = TPE // CH
GC = 40
NG = NCH // GC
NSL = NP // TILES
NP2 = 10112
SPT = NP2 // TILES
ZR = 8

_MESH = plsc.VectorSubcoreMesh(
    core_axis_name="c", subcore_axis_name="s", num_cores=2, num_subcores=16)


@functools.partial(
    pl.kernel,
    out_type=jax.ShapeDtypeStruct((2, NP), jnp.float32),
    mesh=_MESH,
    scratch_types=[
        pltpu.VMEM((TPE,), jnp.int32),
        pltpu.VMEM((NP,), jnp.float32),
        pltpu.VMEM((TILES, NSL), jnp.float32),
        pltpu.VMEM((NSL,), jnp.float32),
        pltpu.VMEM_SHARED((TILES, NP), jnp.float32),
    ],
    compiler_params=pltpu.CompilerParams(needs_layout_passes=False),
)
def _sc_degrees(ei_hbm, deg_hbm, idx_v, cnt_v, tmp_v, acc_v, shared):
    c = lax.axis_index("c")
    s = lax.axis_index("s")
    pltpu.sync_copy(ei_hbm.at[c * TILES + s], idx_v)
    z16 = jnp.zeros((16,), jnp.float32)

    def zbody(j, carry):
        cnt_v[pl.ds(j * 16, 16)] = z16
        return carry
    lax.fori_loop(0, NP // 16, zbody, 0)

    ones = jnp.ones((16,), jnp.float32)

    def abody(j, carry):
        idx = idx_v[pl.ds(j * 16, 16)]
        plsc.addupdate_scatter(cnt_v, [idx], ones)
        return carry
    lax.fori_loop(0, TPE // 16, abody, 0)

    pltpu.sync_copy(cnt_v, shared.at[s])
    plsc.subcore_barrier()
    for r in range(TILES):
        pltpu.sync_copy(shared.at[r, pl.ds(s * NSL, NSL)], tmp_v.at[r])

    def rbody(j, carry):
        v = tmp_v[0, pl.ds(j * 16, 16)]
        for r in range(1, TILES):
            v = v + tmp_v[r, pl.ds(j * 16, 16)]
        acc_v[pl.ds(j * 16, 16)] = v
        return carry
    lax.fori_loop(0, NSL // 16, rbody, 0)
    pltpu.sync_copy(acc_v, deg_hbm.at[c, pl.ds(s * NSL, NSL)])


@functools.partial(
    pl.kernel,
    out_type=jax.ShapeDtypeStruct((2 * NP2, HF), jnp.float32),
    mesh=_MESH,
    scratch_types=[
        pltpu.VMEM((GC, CH), jnp.int32),
        pltpu.VMEM((GC, CH), jnp.int32),
        pltpu.VMEM((2, CH, HF), jnp.float32),
        pltpu.VMEM((ZR, HF), jnp.float32),
        pltpu.VMEM_SHARED((NP2, HF), jnp.float32),
        pltpu.SemaphoreType.DMA((2,)),
        pltpu.SemaphoreType.DMA,
    ],
    compiler_params=pltpu.CompilerParams(needs_layout_passes=False),
)
def _sc_spmm(h_hbm, src_hbm, dst_hbm, agg_hbm, src_m, dst_m, buf, zbuf,
             agg_sh, gsem, zsem):
    c = lax.axis_index("c")
    s = lax.axis_index("s")

    z16 = jnp.zeros((16,), jnp.float32)
    for r in range(ZR):
        for k in range(HF // 16):
            zbuf[r, pl.ds(k * 16, 16)] = z16

    def zc(j, carry):
        pltpu.async_copy(zbuf, agg_sh.at[pl.ds(s * SPT + j * ZR, ZR)], zsem)

        @pl.when(j >= 4)
        def _():
            pltpu.make_async_copy(
                zbuf, agg_sh.at[pl.ds(s * SPT, ZR)], zsem).wait()
        return carry
    lax.fori_loop(0, SPT // ZR, zc, 0)
    for _ in range(4):
        pltpu.make_async_copy(zbuf, agg_sh.at[pl.ds(s * SPT, ZR)],
                              zsem).wait()
    plsc.subcore_barrier()

    for g in range(NG):
        pltpu.sync_copy(src_hbm.at[(c * TILES + s) * NG + g], src_m)
        pltpu.sync_copy(dst_hbm.at[s * NG + g], dst_m)

        pltpu.async_copy(h_hbm.at[src_m.at[0]], buf.at[0], gsem.at[0])

        def body(j, carry):
            p = lax.rem(j, 2)
            q = lax.rem(j + 1, 2)
            pltpu.async_copy(h_hbm.at[src_m.at[j]], buf.at[p], gsem.at[p])
            pltpu.make_async_copy(
                h_hbm.at[src_m.at[j - 1]], buf.at[q], gsem.at[q]).wait()
            pltpu.sync_copy(buf.at[q], agg_sh.at[dst_m.at[j - 1]], add=True)
            return carry
        lax.fori_loop(1, GC, body, 0)

        lp = (GC - 1) % 2
        pltpu.make_async_copy(
            h_hbm.at[src_m.at[GC - 1]], buf.at[lp], gsem.at[lp]).wait()
        pltpu.sync_copy(buf.at[lp], agg_sh.at[dst_m.at[GC - 1]], add=True)

    plsc.subcore_barrier()
    pltpu.sync_copy(agg_sh.at[pl.ds(s * SPT, SPT)],
                    agg_hbm.at[pl.ds(c * NP2 + s * SPT, SPT)])


BN = 1000
GRID = N // BN


def _tc1_body(x_ref, w_ref, dg_ref, out_ref):
    r = lax.rsqrt(jnp.maximum(dg_ref[...], 1.0))
    y = jnp.dot(x_ref[...].astype(jnp.bfloat16),
                w_ref[...].astype(jnp.bfloat16),
                preferred_element_type=jnp.float32)
    y = y * r
    out_ref[0] = y[:, :HF]
    out_ref[1] = y[:, HF:]


_tc1 = pl.pallas_call(
    _tc1_body,
    grid=(GRID,),
    in_specs=[
        pl.BlockSpec((BN, D), lambda i: (i, 0)),
        pl.BlockSpec((D, D), lambda i: (0, 0)),
        pl.BlockSpec((BN, 1), lambda i: (i, 0)),
    ],
    out_specs=pl.BlockSpec((2, BN, HF), lambda i: (0, i, 0)),
    out_shape=jax.ShapeDtypeStruct((2, NP2, HF), jnp.float32),
)


def _tc2_body(a_ref, din_ref, dout_ref, w_ref, b_ref, out_ref):
    rin = lax.rsqrt(jnp.maximum(din_ref[...], 1.0))
    rout = lax.rsqrt(jnp.maximum(dout_ref[...], 1.0))
    a = jnp.concatenate([a_ref[0], a_ref[1]], axis=1)
    h = jnp.maximum(a * rin + b_ref[...], 0.0)
    y = jnp.dot(h.astype(jnp.bfloat16), w_ref[...].astype(jnp.bfloat16),
                preferred_element_type=jnp.float32) * rout
    out_ref[0] = y[:, :HF]
    out_ref[1] = y[:, HF:]


_tc2 = pl.pallas_call(
    _tc2_body,
    grid=(GRID,),
    in_specs=[
        pl.BlockSpec((2, BN, HF), lambda i: (0, i, 0)),
        pl.BlockSpec((BN, 1), lambda i: (i, 0)),
        pl.BlockSpec((BN, 1), lambda i: (i, 0)),
        pl.BlockSpec((D, D), lambda i: (0, 0)),
        pl.BlockSpec((1, D), lambda i: (0, 0)),
    ],
    out_specs=pl.BlockSpec((2, BN, HF), lambda i: (0, i, 0)),
    out_shape=jax.ShapeDtypeStruct((2, NP2, HF), jnp.float32),
)


def _tc3_body(a_ref, din_ref, b_ref, wr_ref, br_ref, out_ref, acc_ref):
    i = pl.program_id(0)

    @pl.when(i == 0)
    def _():
        acc_ref[...] = jnp.zeros_like(acc_ref)

    rin = lax.rsqrt(jnp.maximum(din_ref[...], 1.0))
    a = jnp.concatenate([a_ref[0], a_ref[1]], axis=1)
    h = jnp.maximum(a * rin + b_ref[...], 0.0)
    acc_ref[...] += jnp.sum(h, axis=0, keepdims=True)

    @pl.when(i == GRID - 1)
    def _():
        out_ref[...] = jnp.dot(
            acc_ref[...] * (1.0 / N), wr_ref[...],
            preferred_element_type=jnp.float32) + br_ref[...]


_tc3 = pl.pallas_call(
    _tc3_body,
    grid=(GRID,),
    in_specs=[
        pl.BlockSpec((2, BN, HF), lambda i: (0, i, 0)),
        pl.BlockSpec((BN, 1), lambda i: (i, 0)),
        pl.BlockSpec((1, D), lambda i: (0, 0)),
        pl.BlockSpec((D, OUT), lambda i: (0, 0)),
        pl.BlockSpec((1, OUT), lambda i: (0, 0)),
    ],
    out_specs=pl.BlockSpec((1, OUT), lambda i: (0, 0)),
    out_shape=jax.ShapeDtypeStruct((1, OUT), jnp.float32),
    scratch_shapes=[pltpu.VMEM((1, D), jnp.float32)],
)


def kernel(x, edge_index, W1, b1, W2, b2, Wr, br):
    ei32 = edge_index.reshape(2 * TILES, TPE)
    deg = _sc_degrees(ei32)
    deg_out_col = deg[0, :N].reshape(N, 1)
    deg_in_col = deg[1, :N].reshape(N, 1)

    src = edge_index[0]
    src3 = jnp.stack([src, src + NP2]).reshape(2 * TILES * NG, GC, CH)
    dst3 = edge_index[1].reshape(TILES * NG, GC, CH)

    t1 = _tc1(x, W1, deg_out_col)
    agg1 = _sc_spmm(t1.reshape(2 * NP2, HF), src3, dst3)
    t2 = _tc2(agg1.reshape(2, NP2, HF), deg_in_col, deg_out_col,
              W2, b1.reshape(1, D))
    agg2 = _sc_spmm(t2.reshape(2 * NP2, HF), src3, dst3)
    return _tc3(agg2.reshape(2, NP2, HF), deg_in_col,
                b2.reshape(1, D), Wr, br.reshape(1, OUT))


# speedup vs baseline: 1.4484x; 1.0207x over previous
import functools

import jax
import jax.numpy as jnp
from jax import lax
from jax.experimental import pallas as pl
from jax.experimental.pallas import tpu as pltpu
from jax.experimental.pallas import tpu_sc as plsc

N = 10000
E = 160000
D = 256
OUT = 2
HF = 128
NP = 10240
TILES = 16
TPE = E // TILES
CH = 125
NCH = TPE // CH
GC = 40
NG = NCH // GC
NSL = NP // TILES
NP2 = 10112
SPT = NP2 // TILES
ZR = 8

_MESH = plsc.VectorSubcoreMesh(
    core_axis_name="c", subcore_axis_name="s", num_cores=2, num_subcores=16)


@functools.partial(
    pl.kernel,
    out_type=jax.ShapeDtypeStruct((2, NP), jnp.float32),
    mesh=_MESH,
    scratch_types=[
        pltpu.VMEM((TPE,), jnp.int32),
        pltpu.VMEM((NP,), jnp.float32),
        pltpu.VMEM((TILES, NSL), jnp.float32),
        pltpu.VMEM((NSL,), jnp.float32),
        pltpu.VMEM_SHARED((TILES, NP), jnp.float32),
    ],
    compiler_params=pltpu.CompilerParams(needs_layout_passes=False),
)
def _sc_degrees(ei_hbm, deg_hbm, idx_v, cnt_v, tmp_v, acc_v, shared):
    c = lax.axis_index("c")
    s = lax.axis_index("s")
    pltpu.sync_copy(ei_hbm.at[c * TILES + s], idx_v)
    z16 = jnp.zeros((16,), jnp.float32)

    def zbody(j, carry):
        for u in range(8):
            cnt_v[pl.ds(j * 128 + u * 16, 16)] = z16
        return carry
    lax.fori_loop(0, NP // 128, zbody, 0)

    ones = jnp.ones((16,), jnp.float32)

    def abody(j, carry):
        idx = idx_v[pl.ds(j * 16, 16)]
        plsc.addupdate_scatter(cnt_v, [idx], ones)
        return carry
    lax.fori_loop(0, TPE // 16, abody, 0)

    pltpu.sync_copy(cnt_v, shared.at[s])
    plsc.subcore_barrier()
    for r in range(TILES):
        pltpu.sync_copy(shared.at[r, pl.ds(s * NSL, NSL)], tmp_v.at[r])

    def rbody(j, carry):
        v = tmp_v[0, pl.ds(j * 16, 16)]
        for r in range(1, TILES):
            v = v + tmp_v[r, pl.ds(j * 16, 16)]
        acc_v[pl.ds(j * 16, 16)] = v
        return carry
    lax.fori_loop(0, NSL // 16, rbody, 0)
    pltpu.sync_copy(acc_v, deg_hbm.at[c, pl.ds(s * NSL, NSL)])


@functools.partial(
    pl.kernel,
    out_type=jax.ShapeDtypeStruct((2 * NP2, HF), jnp.float32),
    mesh=_MESH,
    scratch_types=[
        pltpu.VMEM((GC, CH), jnp.int32),
        pltpu.VMEM((GC, CH), jnp.int32),
        pltpu.VMEM((2, CH, HF), jnp.float32),
        pltpu.VMEM((ZR, HF), jnp.float32),
        pltpu.VMEM_SHARED((NP2, HF), jnp.float32),
        pltpu.SemaphoreType.DMA((2,)),
        pltpu.SemaphoreType.DMA,
    ],
    compiler_params=pltpu.CompilerParams(needs_layout_passes=False),
)
def _sc_spmm(h_hbm, src_hbm, dst_hbm, agg_hbm, src_m, dst_m, buf, zbuf,
             agg_sh, gsem, zsem):
    c = lax.axis_index("c")
    s = lax.axis_index("s")

    z16 = jnp.zeros((16,), jnp.float32)
    for r in range(ZR):
        for k in range(HF // 16):
            zbuf[r, pl.ds(k * 16, 16)] = z16

    def zc(j, carry):
        pltpu.async_copy(zbuf, agg_sh.at[pl.ds(s * SPT + j * ZR, ZR)], zsem)

        @pl.when(j >= 4)
        def _():
            pltpu.make_async_copy(
                zbuf, agg_sh.at[pl.ds(s * SPT, ZR)], zsem).wait()
        return carry
    lax.fori_loop(0, SPT // ZR, zc, 0)
    for _ in range(4):
        pltpu.make_async_copy(zbuf, agg_sh.at[pl.ds(s * SPT, ZR)],
                              zsem).wait()
    plsc.subcore_barrier()

    for g in range(NG):
        pltpu.sync_copy(src_hbm.at[(c * TILES + s) * NG + g], src_m)
        pltpu.sync_copy(dst_hbm.at[s * NG + g], dst_m)

        pltpu.async_copy(h_hbm.at[src_m.at[0]], buf.at[0], gsem.at[0])

        def body(j, carry):
            p = lax.rem(j, 2)
            q = lax.rem(j + 1, 2)
            pltpu.async_copy(h_hbm.at[src_m.at[j]], buf.at[p], gsem.at[p])
            pltpu.make_async_copy(
                h_hbm.at[src_m.at[j - 1]], buf.at[q], gsem.at[q]).wait()
            pltpu.sync_copy(buf.at[q], agg_sh.at[dst_m.at[j - 1]], add=True)
            return carry
        lax.fori_loop(1, GC, body, 0)

        lp = (GC - 1) % 2
        pltpu.make_async_copy(
            h_hbm.at[src_m.at[GC - 1]], buf.at[lp], gsem.at[lp]).wait()
        pltpu.sync_copy(buf.at[lp], agg_sh.at[dst_m.at[GC - 1]], add=True)

    plsc.subcore_barrier()
    pltpu.sync_copy(agg_sh.at[pl.ds(s * SPT, SPT)],
                    agg_hbm.at[pl.ds(c * NP2 + s * SPT, SPT)])


BN = 2000
GRID = N // BN


def _tc1_body(x_ref, w_ref, dg_ref, out_ref):
    r = lax.rsqrt(jnp.maximum(dg_ref[...], 1.0))
    y = jnp.dot(x_ref[...].astype(jnp.bfloat16),
                w_ref[...].astype(jnp.bfloat16),
                preferred_element_type=jnp.float32)
    y = y * r
    out_ref[0] = y[:, :HF]
    out_ref[1] = y[:, HF:]


_tc1 = pl.pallas_call(
    _tc1_body,
    grid=(GRID,),
    in_specs=[
        pl.BlockSpec((BN, D), lambda i: (i, 0)),
        pl.BlockSpec((D, D), lambda i: (0, 0)),
        pl.BlockSpec((BN, 1), lambda i: (i, 0)),
    ],
    out_specs=pl.BlockSpec((2, BN, HF), lambda i: (0, i, 0)),
    out_shape=jax.ShapeDtypeStruct((2, NP2, HF), jnp.float32),
)


def _tc2_body(a_ref, din_ref, dout_ref, w_ref, b_ref, out_ref):
    rin = lax.rsqrt(jnp.maximum(din_ref[...], 1.0))
    rout = lax.rsqrt(jnp.maximum(dout_ref[...], 1.0))
    a = jnp.concatenate([a_ref[0], a_ref[1]], axis=1)
    h = jnp.maximum(a * rin + b_ref[...], 0.0)
    y = jnp.dot(h.astype(jnp.bfloat16), w_ref[...].astype(jnp.bfloat16),
                preferred_element_type=jnp.float32) * rout
    out_ref[0] = y[:, :HF]
    out_ref[1] = y[:, HF:]


_tc2 = pl.pallas_call(
    _tc2_body,
    grid=(GRID,),
    in_specs=[
        pl.BlockSpec((2, BN, HF), lambda i: (0, i, 0)),
        pl.BlockSpec((BN, 1), lambda i: (i, 0)),
        pl.BlockSpec((BN, 1), lambda i: (i, 0)),
        pl.BlockSpec((D, D), lambda i: (0, 0)),
        pl.BlockSpec((1, D), lambda i: (0, 0)),
    ],
    out_specs=pl.BlockSpec((2, BN, HF), lambda i: (0, i, 0)),
    out_shape=jax.ShapeDtypeStruct((2, NP2, HF), jnp.float32),
)


def _tc3_body(a_ref, din_ref, b_ref, wr_ref, br_ref, out_ref, acc_ref):
    i = pl.program_id(0)

    @pl.when(i == 0)
    def _():
        acc_ref[...] = jnp.zeros_like(acc_ref)

    rin = lax.rsqrt(jnp.maximum(din_ref[...], 1.0))
    a = jnp.concatenate([a_ref[0], a_ref[1]], axis=1)
    h = jnp.maximum(a * rin + b_ref[...], 0.0)
    acc_ref[...] += jnp.sum(h, axis=0, keepdims=True)

    @pl.when(i == GRID - 1)
    def _():
        out_ref[...] = jnp.dot(
            acc_ref[...] * (1.0 / N), wr_ref[...],
            preferred_element_type=jnp.float32) + br_ref[...]


_tc3 = pl.pallas_call(
    _tc3_body,
    grid=(GRID,),
    in_specs=[
        pl.BlockSpec((2, BN, HF), lambda i: (0, i, 0)),
        pl.BlockSpec((BN, 1), lambda i: (i, 0)),
        pl.BlockSpec((1, D), lambda i: (0, 0)),
        pl.BlockSpec((D, OUT), lambda i: (0, 0)),
        pl.BlockSpec((1, OUT), lambda i: (0, 0)),
    ],
    out_specs=pl.BlockSpec((1, OUT), lambda i: (0, 0)),
    out_shape=jax.ShapeDtypeStruct((1, OUT), jnp.float32),
    scratch_shapes=[pltpu.VMEM((1, D), jnp.float32)],
)


def kernel(x, edge_index, W1, b1, W2, b2, Wr, br):
    ei32 = edge_index.reshape(2 * TILES, TPE)
    deg = _sc_degrees(ei32)
    deg_out_col = deg[0, :N].reshape(N, 1)
    deg_in_col = deg[1, :N].reshape(N, 1)

    src = edge_index[0]
    src3 = jnp.stack([src, src + NP2]).reshape(2 * TILES * NG, GC, CH)
    dst3 = edge_index[1].reshape(TILES * NG, GC, CH)

    t1 = _tc1(x, W1, deg_out_col)
    agg1 = _sc_spmm(t1.reshape(2 * NP2, HF), src3, dst3)
    t2 = _tc2(agg1.reshape(2, NP2, HF), deg_in_col, deg_out_col,
              W2, b1.reshape(1, D))
    agg2 = _sc_spmm(t2.reshape(2 * NP2, HF), src3, dst3)
    return _tc3(agg2.reshape(2, NP2, HF), deg_in_col,
                b2.reshape(1, D), Wr, br.reshape(1, OUT))


# speedup vs baseline: 1.4539x; 1.0038x over previous
import functools

import jax
import jax.numpy as jnp
from jax import lax
from jax.experimental import pallas as pl
from jax.experimental.pallas import tpu as pltpu
from jax.experimental.pallas import tpu_sc as plsc

N = 10000
E = 160000
D = 256
OUT = 2
HF = 128
NP = 10240
TILES = 16
TPE = E // TILES
CH = 125
NCH = TPE // CH
GC = 40
NG = NCH // GC
NSL = NP // TILES
NP2 = 10112
SPT = NP2 // TILES
ZR = 24

_MESH = plsc.VectorSubcoreMesh(
    core_axis_name="c", subcore_axis_name="s", num_cores=2, num_subcores=16)


@functools.partial(
    pl.kernel,
    out_type=jax.ShapeDtypeStruct((2, NP), jnp.float32),
    mesh=_MESH,
    scratch_types=[
        pltpu.VMEM((TPE,), jnp.int32),
        pltpu.VMEM((NP,), jnp.float32),
        pltpu.VMEM((TILES, NSL), jnp.float32),
        pltpu.VMEM((NSL,), jnp.float32),
        pltpu.VMEM_SHARED((TILES, NP), jnp.float32),
    ],
    compiler_params=pltpu.CompilerParams(needs_layout_passes=False),
)
def _sc_degrees(ei_hbm, deg_hbm, idx_v, cnt_v, tmp_v, acc_v, shared):
    c = lax.axis_index("c")
    s = lax.axis_index("s")
    pltpu.sync_copy(ei_hbm.at[c * TILES + s], idx_v)
    z16 = jnp.zeros((16,), jnp.float32)

    def zbody(j, carry):
        for u in range(8):
            cnt_v[pl.ds(j * 128 + u * 16, 16)] = z16
        return carry
    lax.fori_loop(0, NP // 128, zbody, 0)

    ones = jnp.ones((16,), jnp.float32)

    def abody(j, carry):
        idx = idx_v[pl.ds(j * 16, 16)]
        plsc.addupdate_scatter(cnt_v, [idx], ones)
        return carry
    lax.fori_loop(0, TPE // 16, abody, 0)

    pltpu.sync_copy(cnt_v, shared.at[s])
    plsc.subcore_barrier()
    for r in range(TILES):
        pltpu.sync_copy(shared.at[r, pl.ds(s * NSL, NSL)], tmp_v.at[r])

    def rbody(j, carry):
        v = tmp_v[0, pl.ds(j * 16, 16)]
        for r in range(1, TILES):
            v = v + tmp_v[r, pl.ds(j * 16, 16)]
        acc_v[pl.ds(j * 16, 16)] = v
        return carry
    lax.fori_loop(0, NSL // 16, rbody, 0)
    pltpu.sync_copy(acc_v, deg_hbm.at[c, pl.ds(s * NSL, NSL)])


@functools.partial(
    pl.kernel,
    out_type=jax.ShapeDtypeStruct((2 * NP2, HF), jnp.float32),
    mesh=_MESH,
    scratch_types=[
        pltpu.VMEM((GC, CH), jnp.int32),
        pltpu.VMEM((GC, CH), jnp.int32),
        pltpu.VMEM((2, CH, HF), jnp.float32),
        pltpu.VMEM((ZR, HF), jnp.float32),
        pltpu.VMEM_SHARED((NP2, HF), jnp.float32),
        pltpu.SemaphoreType.DMA((2,)),
        pltpu.SemaphoreType.DMA,
    ],
    compiler_params=pltpu.CompilerParams(needs_layout_passes=False),
)
def _sc_spmm(h_hbm, src_hbm, dst_hbm, agg_hbm, src_m, dst_m, buf, zbuf,
             agg_sh, gsem, zsem):
    c = lax.axis_index("c")
    s = lax.axis_index("s")

    z16 = jnp.zeros((16,), jnp.float32)
    for r in range(ZR):
        for k in range(HF // 16):
            zbuf[r, pl.ds(k * 16, 16)] = z16

    nz = SPT // ZR

    def zc(j, carry):
        pltpu.async_copy(zbuf, agg_sh.at[pl.ds(s * SPT + j * ZR, ZR)], zsem)

        @pl.when(j >= 4)
        def _():
            pltpu.make_async_copy(
                zbuf, agg_sh.at[pl.ds(s * SPT, ZR)], zsem).wait()
        return carry
    lax.fori_loop(0, nz, zc, 0)
    ztail = agg_sh.at[pl.ds(s * SPT + nz * ZR, SPT - nz * ZR)]
    pltpu.async_copy(zbuf.at[pl.ds(0, SPT - nz * ZR)], ztail, zsem)

    pltpu.sync_copy(src_hbm.at[(c * TILES + s) * NG], src_m)
    pltpu.sync_copy(dst_hbm.at[s * NG], dst_m)
    pltpu.async_copy(h_hbm.at[src_m.at[0]], buf.at[0], gsem.at[0])

    for _ in range(4):
        pltpu.make_async_copy(zbuf, agg_sh.at[pl.ds(s * SPT, ZR)],
                              zsem).wait()
    pltpu.make_async_copy(zbuf.at[pl.ds(0, SPT - nz * ZR)], ztail,
                          zsem).wait()
    plsc.subcore_barrier()

    for g in range(NG):
        if g > 0:
            pltpu.sync_copy(src_hbm.at[(c * TILES + s) * NG + g], src_m)
            pltpu.sync_copy(dst_hbm.at[s * NG + g], dst_m)
            pltpu.async_copy(h_hbm.at[src_m.at[0]], buf.at[0], gsem.at[0])

        def body(j, carry):
            p = lax.rem(j, 2)
            q = lax.rem(j + 1, 2)
            pltpu.async_copy(h_hbm.at[src_m.at[j]], buf.at[p], gsem.at[p])
            pltpu.make_async_copy(
                h_hbm.at[src_m.at[j - 1]], buf.at[q], gsem.at[q]).wait()
            pltpu.sync_copy(buf.at[q], agg_sh.at[dst_m.at[j - 1]], add=True)
            return carry
        lax.fori_loop(1, GC, body, 0)

        lp = (GC - 1) % 2
        pltpu.make_async_copy(
            h_hbm.at[src_m.at[GC - 1]], buf.at[lp], gsem.at[lp]).wait()
        pltpu.sync_copy(buf.at[lp], agg_sh.at[dst_m.at[GC - 1]], add=True)

    plsc.subcore_barrier()
    pltpu.sync_copy(agg_sh.at[pl.ds(s * SPT, SPT)],
                    agg_hbm.at[pl.ds(c * NP2 + s * SPT, SPT)])


BN = 2000
GRID = N // BN


def _tc1_body(x_ref, w_ref, dg_ref, out_ref):
    r = lax.rsqrt(jnp.maximum(dg_ref[...], 1.0))
    y = jnp.dot(x_ref[...].astype(jnp.bfloat16),
                w_ref[...].astype(jnp.bfloat16),
                preferred_element_type=jnp.float32)
    y = y * r
    out_ref[0] = y[:, :HF]
    out_ref[1] = y[:, HF:]


_tc1 = pl.pallas_call(
    _tc1_body,
    grid=(GRID,),
    in_specs=[
        pl.BlockSpec((BN, D), lambda i: (i, 0)),
        pl.BlockSpec((D, D), lambda i: (0, 0)),
        pl.BlockSpec((BN, 1), lambda i: (i, 0)),
    ],
    out_specs=pl.BlockSpec((2, BN, HF), lambda i: (0, i, 0)),
    out_shape=jax.ShapeDtypeStruct((2, NP2, HF), jnp.float32),
)


def _tc2_body(a_ref, din_ref, dout_ref, w_ref, b_ref, out_ref):
    rin = lax.rsqrt(jnp.maximum(din_ref[...], 1.0))
    rout = lax.rsqrt(jnp.maximum(dout_ref[...], 1.0))
    a = jnp.concatenate([a_ref[0], a_ref[1]], axis=1)
    h = jnp.maximum(a * rin + b_ref[...], 0.0)
    y = jnp.dot(h.astype(jnp.bfloat16), w_ref[...].astype(jnp.bfloat16),
                preferred_element_type=jnp.float32) * rout
    out_ref[0] = y[:, :HF]
    out_ref[1] = y[:, HF:]


_tc2 = pl.pallas_call(
    _tc2_body,
    grid=(GRID,),
    in_specs=[
        pl.BlockSpec((2, BN, HF), lambda i: (0, i, 0)),
        pl.BlockSpec((BN, 1), lambda i: (i, 0)),
        pl.BlockSpec((BN, 1), lambda i: (i, 0)),
        pl.BlockSpec((D, D), lambda i: (0, 0)),
        pl.BlockSpec((1, D), lambda i: (0, 0)),
    ],
    out_specs=pl.BlockSpec((2, BN, HF), lambda i: (0, i, 0)),
    out_shape=jax.ShapeDtypeStruct((2, NP2, HF), jnp.float32),
)


def _tc3_body(a_ref, din_ref, b_ref, wr_ref, br_ref, out_ref, acc_ref):
    i = pl.program_id(0)

    @pl.when(i == 0)
    def _():
        acc_ref[...] = jnp.zeros_like(acc_ref)

    rin = lax.rsqrt(jnp.maximum(din_ref[...], 1.0))
    a = jnp.concatenate([a_ref[0], a_ref[1]], axis=1)
    h = jnp.maximum(a * rin + b_ref[...], 0.0)
    acc_ref[...] += jnp.sum(h, axis=0, keepdims=True)

    @pl.when(i == GRID - 1)
    def _():
        out_ref[...] = jnp.dot(
            acc_ref[...] * (1.0 / N), wr_ref[...],
            preferred_element_type=jnp.float32) + br_ref[...]


_tc3 = pl.pallas_call(
    _tc3_body,
    grid=(GRID,),
    in_specs=[
        pl.BlockSpec((2, BN, HF), lambda i: (0, i, 0)),
        pl.BlockSpec((BN, 1), lambda i: (i, 0)),
        pl.BlockSpec((1, D), lambda i: (0, 0)),
        pl.BlockSpec((D, OUT), lambda i: (0, 0)),
        pl.BlockSpec((1, OUT), lambda i: (0, 0)),
    ],
    out_specs=pl.BlockSpec((1, OUT), lambda i: (0, 0)),
    out_shape=jax.ShapeDtypeStruct((1, OUT), jnp.float32),
    scratch_shapes=[pltpu.VMEM((1, D), jnp.float32)],
)


def kernel(x, edge_index, W1, b1, W2, b2, Wr, br):
    ei32 = edge_index.reshape(2 * TILES, TPE)
    deg = _sc_degrees(ei32)
    deg_out_col = deg[0, :N].reshape(N, 1)
    deg_in_col = deg[1, :N].reshape(N, 1)

    src = edge_index[0]
    src3 = jnp.stack([src, src + NP2]).reshape(2 * TILES * NG, GC, CH)
    dst3 = edge_index[1].reshape(TILES * NG, GC, CH)

    t1 = _tc1(x, W1, deg_out_col)
    agg1 = _sc_spmm(t1.reshape(2 * NP2, HF), src3, dst3)
    t2 = _tc2(agg1.reshape(2, NP2, HF), deg_in_col, deg_out_col,
              W2, b1.reshape(1, D))
    agg2 = _sc_spmm(t2.reshape(2 * NP2, HF), src3, dst3)
    return _tc3(agg2.reshape(2, NP2, HF), deg_in_col,
                b2.reshape(1, D), Wr, br.reshape(1, OUT))
